# Initial kernel scaffold; baseline (speedup 1.0000x reference)
import jax
import jax.numpy as jnp
from jax.experimental import pallas as pl


def kernel(indices, pos):
    raise NotImplementedError("write your pallas kernel here")


# speedup vs baseline: 4.1549x; 4.1549x over previous
import functools

import jax
import jax.numpy as jnp
from jax import lax
from jax.experimental import pallas as pl
from jax.experimental.pallas import tpu as pltpu
from jax.experimental.pallas import tpu_sc as plsc

_D = 64
_B = 4096 * 200
_NC, _NS = 2, 16
_NW = _NC * _NS
_BPW = _B // _NW
_IW = 128
_G = 8
_CHUNK = _G * _IW
_NCHUNK = _BPW // _CHUNK


def _sc_gather(table, idx2d):
    mesh = plsc.VectorSubcoreMesh(core_axis_name="c", subcore_axis_name="s")

    @functools.partial(
        pl.kernel,
        mesh=mesh,
        out_type=jax.ShapeDtypeStruct((_B, _D), jnp.float32),
        scratch_types=[
            pltpu.VMEM((_G, _IW), jnp.int32),
            pltpu.VMEM((_CHUNK, _D), jnp.float32),
            pltpu.SemaphoreType.DMA,
        ],
        compiler_params=pltpu.CompilerParams(use_tc_tiling_on_sc=False),
    )
    def k(table_hbm, idx_hbm, out_hbm, idx_v, rows_v, sem):
        wid = lax.axis_index("s") * _NC + lax.axis_index("c")
        base = wid * _BPW

        def chunk(i, carry):
            off = pl.multiple_of(base + i * _CHUNK, _CHUNK)
            row0 = pl.multiple_of(off // _IW, _G)
            pltpu.sync_copy(idx_hbm.at[pl.ds(row0, _G)], idx_v)
            cps = [
                pltpu.async_copy(
                    table_hbm.at[idx_v.at[j]],
                    rows_v.at[pl.ds(j * _IW, _IW)],
                    sem,
                )
                for j in range(_G)
            ]
            for cp in cps:
                cp.wait()
            pltpu.sync_copy(rows_v, out_hbm.at[pl.ds(off, _CHUNK)])
            return carry

        lax.fori_loop(0, _NCHUNK, chunk, 0)

    return k(table, idx2d)


def kernel(indices, pos):
    table = pos[0]
    idx2d = indices.reshape(_B // _IW, _IW).astype(jnp.int32)
    out = _sc_gather(table, idx2d)
    return out.reshape(indices.shape[0], indices.shape[1], _D)


# speedup vs baseline: 4.1877x; 1.0079x over previous
import functools

import jax
import jax.numpy as jnp
from jax import lax
from jax.experimental import pallas as pl
from jax.experimental.pallas import tpu as pltpu
from jax.experimental.pallas import tpu_sc as plsc

_D = 64
_B = 4096 * 200
_NC, _NS = 2, 16
_NW = _NC * _NS
_BPW = _B // _NW
_IW = 128
_G = 5
_CHUNK = _G * _IW
_NCHUNK = _BPW // _CHUNK


def _sc_gather(table, idx2d):
    mesh = plsc.VectorSubcoreMesh(core_axis_name="c", subcore_axis_name="s")

    @functools.partial(
        pl.kernel,
        mesh=mesh,
        out_type=jax.ShapeDtypeStruct((_B, _D), jnp.float32),
        scratch_types=[
            pltpu.VMEM((2, _G, _IW), jnp.int32),
            pltpu.VMEM((2, _CHUNK, _D), jnp.float32),
            pltpu.SemaphoreType.DMA,
            pltpu.SemaphoreType.DMA,
        ],
        compiler_params=pltpu.CompilerParams(use_tc_tiling_on_sc=False),
    )
    def k(table_hbm, idx_hbm, out_hbm, idx_v, rows_v, sem_g, sem_out):
        wid = lax.axis_index("s") * _NC + lax.axis_index("c")
        base = wid * _BPW

        def do_chunk(g, ib, first):
            off = pl.multiple_of(base + g * _CHUNK, _CHUNK)
            row0 = pl.multiple_of(off // _IW, _G)
            pltpu.sync_copy(idx_hbm.at[pl.ds(row0, _G)], idx_v.at[ib])
            if not first:
                pltpu.make_async_copy(
                    rows_v.at[ib], out_hbm.at[pl.ds(0, _CHUNK)], sem_out
                ).wait()
            cps = [
                pltpu.async_copy(
                    table_hbm.at[idx_v.at[ib].at[j]],
                    rows_v.at[ib].at[pl.ds(j * _IW, _IW)],
                    sem_g,
                )
                for j in range(_G)
            ]
            for cp in cps:
                cp.wait()
            pltpu.async_copy(rows_v.at[ib], out_hbm.at[pl.ds(off, _CHUNK)], sem_out)

        do_chunk(0, 0, True)
        do_chunk(1, 1, True)

        def pair(kk, carry):
            g0 = pl.multiple_of(kk * 2, 2)
            do_chunk(g0, 0, False)
            do_chunk(g0 + 1, 1, False)
            return carry

        lax.fori_loop(1, _NCHUNK // 2, pair, 0)

        for ib in range(2):
            pltpu.make_async_copy(
                rows_v.at[ib], out_hbm.at[pl.ds(0, _CHUNK)], sem_out
            ).wait()

    return k(table, idx2d)


def kernel(indices, pos):
    table = pos[0]
    idx2d = indices.reshape(_B // _IW, _IW).astype(jnp.int32)
    out = _sc_gather(table, idx2d)
    return out.reshape(indices.shape[0], indices.shape[1], _D)
